# Initial kernel scaffold; baseline (speedup 1.0000x reference)
import jax
import jax.numpy as jnp
from jax.experimental import pallas as pl


def kernel(x, indices):
    raise NotImplementedError("write your pallas kernel here")


# speedup vs baseline: 47.7998x; 47.7998x over previous
import functools

import jax
import jax.numpy as jnp
from jax import lax
from jax.experimental import pallas as pl
from jax.experimental.pallas import tpu as pltpu
from jax.experimental.pallas import tpu_sc as plsc

_NC = 2
_NS = 16
_L = 16


def kernel(x, indices):
    B, C, H, W = x.shape
    Ho, Wo = 2 * H, 2 * W
    P = B * C
    N = H * W
    M = Ho * Wo
    NWORK = _NC * _NS
    PPW = P // NWORK
    NSTRIP = 4
    SLEN = M // NSTRIP
    UN = 8

    assert P % NWORK == 0 and M % NSTRIP == 0
    assert N % (_L * UN) == 0 and SLEN % (_L * UN) == 0

    xf = x.reshape(P, N)
    idxf = indices.reshape(P, N)

    mesh = plsc.VectorSubcoreMesh(core_axis_name="c", subcore_axis_name="s")

    @functools.partial(
        pl.kernel,
        out_type=jax.ShapeDtypeStruct((P, NSTRIP, SLEN), jnp.float32),
        mesh=mesh,
        compiler_params=pltpu.CompilerParams(needs_layout_passes=False),
        scratch_types=[
            pltpu.VMEM((N,), jnp.float32),
            pltpu.VMEM((N,), jnp.int32),
            pltpu.VMEM((SLEN,), jnp.float32),
        ],
    )
    def unpool(x_hbm, idx_hbm, out_hbm, x_v, idx_v, strip_v):
        cid = lax.axis_index("c")
        sid = lax.axis_index("s")
        wid = sid * _NC + cid

        zeros = jnp.zeros((_L,), jnp.float32)

        def plane_body(j, carry):
            p = wid * PPW + j
            pltpu.sync_copy(x_hbm.at[p], x_v)
            pltpu.sync_copy(idx_hbm.at[p], idx_v)
            for s in range(NSTRIP):
                base = s * SLEN

                def zbody(i, c):
                    for u in range(UN):
                        strip_v[pl.ds((i * UN + u) * _L, _L)] = zeros
                    return c

                lax.fori_loop(0, SLEN // (_L * UN), zbody, 0, unroll=False)

                def sbody(i, c):
                    for u in range(UN):
                        off = (i * UN + u) * _L
                        iv = idx_v[pl.ds(off, _L)]
                        xv = x_v[pl.ds(off, _L)]
                        t = iv - base
                        m = (iv >= base) & (iv < base + SLEN)
                        plsc.store_scatter(strip_v, [t], xv, mask=m)
                    return c

                lax.fori_loop(0, N // (_L * UN), sbody, 0, unroll=False)
                pltpu.sync_copy(strip_v, out_hbm.at[p, s])
            return carry

        lax.fori_loop(0, PPW, plane_body, 0, unroll=False)

    out = unpool(xf, idxf)
    return out.reshape(B, C, Ho, Wo)


# speedup vs baseline: 48.1914x; 1.0082x over previous
import functools

import jax
import jax.numpy as jnp
from jax import lax
from jax.experimental import pallas as pl
from jax.experimental.pallas import tpu as pltpu
from jax.experimental.pallas import tpu_sc as plsc

_NC = 2
_NS = 16
_L = 16


def kernel(x, indices):
    B, C, H, W = x.shape
    Ho, Wo = 2 * H, 2 * W
    P = B * C
    N = H * W
    M = Ho * Wo
    NWORK = _NC * _NS
    PPW = P // NWORK
    NSTRIP = 4
    SLEN = M // NSTRIP
    UN = 16

    assert P % NWORK == 0 and M % NSTRIP == 0
    assert N % (_L * UN) == 0 and SLEN % (_L * UN) == 0

    xf = x.reshape(P, N)
    idxf = indices.reshape(P, N)

    mesh = plsc.VectorSubcoreMesh(core_axis_name="c", subcore_axis_name="s")

    @functools.partial(
        pl.kernel,
        out_type=jax.ShapeDtypeStruct((P, NSTRIP, SLEN), jnp.float32),
        mesh=mesh,
        compiler_params=pltpu.CompilerParams(needs_layout_passes=False),
        scratch_types=[
            pltpu.VMEM((N,), jnp.float32),
            pltpu.VMEM((N,), jnp.int32),
            pltpu.VMEM((SLEN,), jnp.float32),
        ],
    )
    def unpool(x_hbm, idx_hbm, out_hbm, x_v, idx_v, strip_v):
        cid = lax.axis_index("c")
        sid = lax.axis_index("s")
        wid = sid * _NC + cid

        zeros = jnp.zeros((_L,), jnp.float32)

        def plane_body(j, carry):
            p = wid * PPW + j
            pltpu.sync_copy(x_hbm.at[p], x_v)
            pltpu.sync_copy(idx_hbm.at[p], idx_v)
            for s in range(NSTRIP):
                base = s * SLEN

                def zbody(i, c):
                    for u in range(UN):
                        strip_v[pl.ds((i * UN + u) * _L, _L)] = zeros
                    return c

                lax.fori_loop(0, SLEN // (_L * UN), zbody, 0, unroll=False)

                def sbody(i, c):
                    for u in range(UN):
                        off = (i * UN + u) * _L
                        iv = idx_v[pl.ds(off, _L)]
                        xv = x_v[pl.ds(off, _L)]
                        t = iv - base
                        m = plsc.bitcast(t, jnp.uint32) < jnp.uint32(SLEN)
                        plsc.store_scatter(strip_v, [t], xv, mask=m)
                    return c

                lax.fori_loop(0, N // (_L * UN), sbody, 0, unroll=False)
                pltpu.sync_copy(strip_v, out_hbm.at[p, s])
            return carry

        lax.fori_loop(0, PPW, plane_body, 0, unroll=False)

    out = unpool(xf, idxf)
    return out.reshape(B, C, Ho, Wo)
